# baseline (speedup 1.0000x reference)
import functools

import jax
import jax.numpy as jnp
from jax import lax
from jax.experimental import pallas as pl
from jax.experimental.pallas import tpu as pltpu
from jax.experimental.pallas import tpu_sc as plsc

Q = 1024
K = 100000
D = 64
TOPK = 10

QT = 128
BK = 2048
NB = (K + BK - 1) // BK
KPAD = NB * BK
SB = 256
SPB = BK // SB
NSB = KPAD // SB
CW = 16
GC = TOPK * SB

NEG_INF = float("-inf")
I32_MAX = 2**31 - 1


def _score_max_kernel(q_ref, k_ref, s_ref, mx_ref):
    bi = pl.program_id(0)
    q = q_ref[...]
    kb = k_ref[...]
    s = lax.dot_general(q, kb, (((1,), (1,)), ((), ())),
                        preferred_element_type=jnp.float32)
    col = lax.broadcasted_iota(jnp.int32, (QT, BK), 1)
    s = jnp.where(col + bi * BK < K, s, NEG_INF)
    mxs = []
    for j in range(SPB):
        sj = s[:, j * SB:(j + 1) * SB]
        s_ref[:, j, :] = sj
        mxs.append(jnp.max(sj, axis=1, keepdims=True))
    mx_ref[0] = jnp.concatenate(mxs, axis=1)


def _select_blocks_kernel(mx_ref, sb_ref, row_ref):
    qi = pl.program_id(0)
    c = jnp.concatenate([mx_ref[b] for b in range(NB)], axis=1)
    g = lax.broadcasted_iota(jnp.int32, (QT, NSB), 1)
    qrow = (lax.broadcasted_iota(jnp.int32, (QT, 1), 0) + qi * QT) * NSB
    out_sb, out_row = [], []
    for _ in range(TOPK):
        m = jnp.max(c, axis=1, keepdims=True)
        hit = c == m
        a = jnp.min(jnp.where(hit, g, I32_MAX), axis=1, keepdims=True)
        out_sb.append(a)
        out_row.append(qrow + a)
        c = jnp.where(g == a, NEG_INF, c)
    pad = jnp.zeros((QT, CW - TOPK), jnp.int32)
    sb_ref[...] = jnp.concatenate(out_sb + [pad], axis=1)
    row_ref[...] = jnp.concatenate(out_row + [pad], axis=1)


def _final_topk_kernel(gs_ref, sb_ref, os_ref, oi_ref):
    c = gs_ref[...]
    sb = sb_ref[...]
    off = lax.broadcasted_iota(jnp.int32, (QT, SB), 1)
    g = jnp.concatenate(
        [sb[:, j:j + 1] * SB + off for j in range(TOPK)], axis=1)
    out_s, out_i = [], []
    for _ in range(TOPK):
        m = jnp.max(c, axis=1, keepdims=True)
        hit = c == m
        a = jnp.min(jnp.where(hit, g, I32_MAX), axis=1, keepdims=True)
        out_s.append(m)
        out_i.append(a)
        c = jnp.where(hit & (g == a), NEG_INF, c)
    pad_s = jnp.full((QT, CW - TOPK), NEG_INF, jnp.float32)
    pad_i = jnp.zeros((QT, CW - TOPK), jnp.int32)
    os_ref[...] = jnp.concatenate(out_s + [pad_s], axis=1)
    oi_ref[...] = jnp.concatenate(out_i + [pad_i], axis=1)


def _sc_gather(table, idx, width):
    info = plsc.get_sparse_core_info()
    nw = info.num_cores * info.num_subcores
    b = idx.shape[0]
    bpw = b // nw
    mesh = plsc.VectorSubcoreMesh(core_axis_name="c", subcore_axis_name="s")

    @functools.partial(
        pl.kernel,
        mesh=mesh,
        out_type=jax.ShapeDtypeStruct((b, width), jnp.float32),
        compiler_params=pltpu.CompilerParams(use_tc_tiling_on_sc=False),
        scratch_types=[
            pltpu.VMEM((bpw,), jnp.int32),
            pltpu.VMEM((bpw, width), jnp.float32),
            pltpu.SemaphoreType.DMA,
        ],
    )
    def gather_k(table_hbm, idx_hbm, out_hbm, idx_v, rows_v, sem):
        wid = lax.axis_index("s") * info.num_cores + lax.axis_index("c")
        base = wid * bpw
        pltpu.sync_copy(idx_hbm.at[pl.ds(base, bpw)], idx_v)
        pltpu.async_copy(table_hbm.at[idx_v], rows_v, sem).wait()
        pltpu.sync_copy(rows_v, out_hbm.at[pl.ds(base, bpw)])

    return gather_k(table, idx)


def kernel(query, key_memory, value_memory, k):
    nq = Q // QT
    scores, mx = pl.pallas_call(
        _score_max_kernel,
        grid=(NB, nq),
        in_specs=[
            pl.BlockSpec((QT, D), lambda bi, qi: (qi, 0)),
            pl.BlockSpec((BK, D), lambda bi, qi: (bi, 0)),
        ],
        out_specs=[
            pl.BlockSpec((QT, SPB, SB), lambda bi, qi: (qi, bi, 0)),
            pl.BlockSpec((1, QT, SPB), lambda bi, qi: (bi, qi, 0)),
        ],
        out_shape=[
            jax.ShapeDtypeStruct((Q, NSB, SB), jnp.float32),
            jax.ShapeDtypeStruct((NB, Q, SPB), jnp.float32),
        ],
    )(query, key_memory)

    sb, rows = pl.pallas_call(
        _select_blocks_kernel,
        grid=(nq,),
        in_specs=[pl.BlockSpec((NB, QT, SPB), lambda qi: (0, qi, 0))],
        out_specs=[
            pl.BlockSpec((QT, CW), lambda qi: (qi, 0)),
            pl.BlockSpec((QT, CW), lambda qi: (qi, 0)),
        ],
        out_shape=[
            jax.ShapeDtypeStruct((Q, CW), jnp.int32),
            jax.ShapeDtypeStruct((Q, CW), jnp.int32),
        ],
    )(mx)

    gathered = _sc_gather(scores.reshape(Q * NSB, SB),
                          rows[:, :TOPK].reshape(-1), SB)

    ts, ti = pl.pallas_call(
        _final_topk_kernel,
        grid=(nq,),
        in_specs=[
            pl.BlockSpec((QT, GC), lambda qi: (qi, 0)),
            pl.BlockSpec((QT, CW), lambda qi: (qi, 0)),
        ],
        out_specs=[
            pl.BlockSpec((QT, CW), lambda qi: (qi, 0)),
            pl.BlockSpec((QT, CW), lambda qi: (qi, 0)),
        ],
        out_shape=[
            jax.ShapeDtypeStruct((Q, CW), jnp.float32),
            jax.ShapeDtypeStruct((Q, CW), jnp.int32),
        ],
    )(gathered.reshape(Q, GC), sb)

    rows_out = _sc_gather(value_memory, ti[:, :TOPK].reshape(-1), D)
    return rows_out.reshape(Q, TOPK, D), ts[:, :TOPK]


# speedup vs baseline: 1.4058x; 1.4058x over previous
import functools

import jax
import jax.numpy as jnp
from jax import lax
from jax.experimental import pallas as pl
from jax.experimental.pallas import tpu as pltpu
from jax.experimental.pallas import tpu_sc as plsc

Q = 1024
K = 100000
D = 64
TOPK = 10

QT = 128
BK = 2048
NB = (K + BK - 1) // BK
KPAD = NB * BK
SB = 128
SPB = BK // SB
NSB = KPAD // SB
CW = 16
GC = TOPK * SB

NEG_INF = float("-inf")
I32_MAX = 2**31 - 1


def _score_max_kernel(q_ref, k_ref, s_ref, mx_ref):
    bi = pl.program_id(0)
    q = q_ref[...]
    kb = k_ref[...]
    s = lax.dot_general(q, kb, (((1,), (1,)), ((), ())),
                        preferred_element_type=jnp.float32)
    col = lax.broadcasted_iota(jnp.int32, (QT, BK), 1)
    s = jnp.where(col + bi * BK < K, s, NEG_INF)
    mxs = []
    for j in range(SPB):
        sj = s[:, j * SB:(j + 1) * SB]
        s_ref[:, j, :] = sj
        mxs.append(jnp.max(sj, axis=1, keepdims=True))
    mx_ref[0] = jnp.concatenate(mxs, axis=1)


def _select_blocks_kernel(mx_ref, sb_ref, row_ref):
    qi = pl.program_id(0)
    c = jnp.concatenate([mx_ref[b] for b in range(NB)], axis=1)
    g = lax.broadcasted_iota(jnp.int32, (QT, NSB), 1)
    qrow = (lax.broadcasted_iota(jnp.int32, (QT, 1), 0) + qi * QT) * NSB
    out_sb, out_row = [], []
    for _ in range(TOPK):
        m = jnp.max(c, axis=1, keepdims=True)
        hit = c == m
        a = jnp.min(jnp.where(hit, g, I32_MAX), axis=1, keepdims=True)
        out_sb.append(a)
        out_row.append(qrow + a)
        c = jnp.where(g == a, NEG_INF, c)
    pad = jnp.zeros((QT, CW - TOPK), jnp.int32)
    sb_ref[...] = jnp.concatenate(out_sb + [pad], axis=1)
    row_ref[...] = jnp.concatenate(out_row + [pad], axis=1)


def _final_topk_kernel(gs_ref, sb_ref, os_ref, oi_ref):
    c = gs_ref[...]
    sb = sb_ref[...]
    off = lax.broadcasted_iota(jnp.int32, (QT, SB), 1)
    g = jnp.concatenate(
        [sb[:, j:j + 1] * SB + off for j in range(TOPK)], axis=1)
    out_s, out_i = [], []
    for _ in range(TOPK):
        m = jnp.max(c, axis=1, keepdims=True)
        hit = c == m
        a = jnp.min(jnp.where(hit, g, I32_MAX), axis=1, keepdims=True)
        out_s.append(m)
        out_i.append(a)
        c = jnp.where(hit & (g == a), NEG_INF, c)
    pad_s = jnp.full((QT, CW - TOPK), NEG_INF, jnp.float32)
    pad_i = jnp.zeros((QT, CW - TOPK), jnp.int32)
    os_ref[...] = jnp.concatenate(out_s + [pad_s], axis=1)
    oi_ref[...] = jnp.concatenate(out_i + [pad_i], axis=1)


def _sc_gather(table, idx, width):
    info = plsc.get_sparse_core_info()
    nw = info.num_cores * info.num_subcores
    b = idx.shape[0]
    bpw = b // nw
    mesh = plsc.VectorSubcoreMesh(core_axis_name="c", subcore_axis_name="s")

    @functools.partial(
        pl.kernel,
        mesh=mesh,
        out_type=jax.ShapeDtypeStruct((b, width), jnp.float32),
        compiler_params=pltpu.CompilerParams(use_tc_tiling_on_sc=False),
        scratch_types=[
            pltpu.VMEM((bpw,), jnp.int32),
            pltpu.VMEM((bpw, width), jnp.float32),
            pltpu.SemaphoreType.DMA,
        ],
    )
    def gather_k(table_hbm, idx_hbm, out_hbm, idx_v, rows_v, sem):
        wid = lax.axis_index("s") * info.num_cores + lax.axis_index("c")
        base = wid * bpw
        pltpu.sync_copy(idx_hbm.at[pl.ds(base, bpw)], idx_v)
        pltpu.async_copy(table_hbm.at[idx_v], rows_v, sem).wait()
        pltpu.sync_copy(rows_v, out_hbm.at[pl.ds(base, bpw)])

    return gather_k(table, idx)


def kernel(query, key_memory, value_memory, k):
    nq = Q // QT
    scores, mx = pl.pallas_call(
        _score_max_kernel,
        grid=(NB, nq),
        in_specs=[
            pl.BlockSpec((QT, D), lambda bi, qi: (qi, 0)),
            pl.BlockSpec((BK, D), lambda bi, qi: (bi, 0)),
        ],
        out_specs=[
            pl.BlockSpec((QT, SPB, SB), lambda bi, qi: (qi, bi, 0)),
            pl.BlockSpec((1, QT, SPB), lambda bi, qi: (bi, qi, 0)),
        ],
        out_shape=[
            jax.ShapeDtypeStruct((Q, NSB, SB), jnp.float32),
            jax.ShapeDtypeStruct((NB, Q, SPB), jnp.float32),
        ],
    )(query, key_memory)

    sb, rows = pl.pallas_call(
        _select_blocks_kernel,
        grid=(nq,),
        in_specs=[pl.BlockSpec((NB, QT, SPB), lambda qi: (0, qi, 0))],
        out_specs=[
            pl.BlockSpec((QT, CW), lambda qi: (qi, 0)),
            pl.BlockSpec((QT, CW), lambda qi: (qi, 0)),
        ],
        out_shape=[
            jax.ShapeDtypeStruct((Q, CW), jnp.int32),
            jax.ShapeDtypeStruct((Q, CW), jnp.int32),
        ],
    )(mx)

    gathered = _sc_gather(scores.reshape(Q * NSB, SB),
                          rows[:, :TOPK].reshape(-1), SB)

    ts, ti = pl.pallas_call(
        _final_topk_kernel,
        grid=(nq,),
        in_specs=[
            pl.BlockSpec((QT, GC), lambda qi: (qi, 0)),
            pl.BlockSpec((QT, CW), lambda qi: (qi, 0)),
        ],
        out_specs=[
            pl.BlockSpec((QT, CW), lambda qi: (qi, 0)),
            pl.BlockSpec((QT, CW), lambda qi: (qi, 0)),
        ],
        out_shape=[
            jax.ShapeDtypeStruct((Q, CW), jnp.float32),
            jax.ShapeDtypeStruct((Q, CW), jnp.int32),
        ],
    )(gathered.reshape(Q, GC), sb)

    rows_out = _sc_gather(value_memory, ti[:, :TOPK].reshape(-1), D)
    return rows_out.reshape(Q, TOPK, D), ts[:, :TOPK]


# speedup vs baseline: 1.5335x; 1.0908x over previous
import functools

import jax
import jax.numpy as jnp
from jax import lax
from jax.experimental import pallas as pl
from jax.experimental.pallas import tpu as pltpu
from jax.experimental.pallas import tpu_sc as plsc

Q = 1024
K = 100000
D = 64
TOPK = 10

QT = 128
QH = QT // 8
BK = 2048
NB = (K + BK - 1) // BK
KPAD = NB * BK
SB = 128
SPB = BK // SB
NSB = KPAD // SB
CW = 16
GC = TOPK * SB

NEG_INF = float("-inf")
I32_MAX = 2**31 - 1


def _score_max_kernel(q_ref, k_ref, s_ref, mx_ref):
    bi = pl.program_id(0)
    q = q_ref[...]
    kb = k_ref[...]
    s = lax.dot_general(q, kb, (((1,), (1,)), ((), ())),
                        preferred_element_type=jnp.float32)
    col = lax.broadcasted_iota(jnp.int32, (QT, BK), 1)
    s = jnp.where(col + bi * BK < K, s, NEG_INF)
    mxs = []
    for j in range(SPB):
        sj = s[:, j * SB:(j + 1) * SB]
        s_ref[:, j] = sj.reshape(QH, 8, SB)
        mxs.append(jnp.max(sj, axis=1, keepdims=True))
    mx_ref[0] = jnp.concatenate(mxs, axis=1)


def _select_blocks_kernel(mx_ref, sb_ref, row_ref):
    qi = pl.program_id(0)
    c = jnp.concatenate([mx_ref[b] for b in range(NB)], axis=1)
    g = lax.broadcasted_iota(jnp.int32, (QT, NSB), 1)
    qglob = lax.broadcasted_iota(jnp.int32, (QT, 1), 0) + qi * QT
    rbase = (qglob >> 3) * (NSB * 8) + (qglob & 7)
    out_sb, out_row = [], []
    for _ in range(TOPK):
        m = jnp.max(c, axis=1, keepdims=True)
        hit = c == m
        a = jnp.min(jnp.where(hit, g, I32_MAX), axis=1, keepdims=True)
        out_sb.append(a)
        out_row.append(rbase + a * 8)
        c = jnp.where(g == a, NEG_INF, c)
    pad = jnp.zeros((QT, CW - TOPK), jnp.int32)
    sb_ref[...] = jnp.concatenate(out_sb + [pad], axis=1)
    row_ref[...] = jnp.concatenate(out_row + [pad], axis=1)


def _final_topk_kernel(g4_ref, sb_ref, os_ref, oi_ref):
    c = jnp.concatenate(
        [g4_ref[:, j].reshape(QT, SB) for j in range(TOPK)], axis=1)
    sb = sb_ref[...]
    off = lax.broadcasted_iota(jnp.int32, (QT, SB), 1)
    g = jnp.concatenate(
        [sb[:, j:j + 1] * SB + off for j in range(TOPK)], axis=1)
    out_s, out_i = [], []
    for _ in range(TOPK):
        m = jnp.max(c, axis=1, keepdims=True)
        hit = c == m
        a = jnp.min(jnp.where(hit, g, I32_MAX), axis=1, keepdims=True)
        out_s.append(m)
        out_i.append(a)
        c = jnp.where(hit & (g == a), NEG_INF, c)
    pad_s = jnp.full((QT, CW - TOPK), NEG_INF, jnp.float32)
    pad_i = jnp.zeros((QT, CW - TOPK), jnp.int32)
    os_ref[...] = jnp.concatenate(out_s + [pad_s], axis=1)
    oi_ref[...] = jnp.concatenate(out_i + [pad_i], axis=1)


def _sc_gather(table, idx, width):
    info = plsc.get_sparse_core_info()
    nw = info.num_cores * info.num_subcores
    b = idx.shape[0]
    bpw = b // nw
    mesh = plsc.VectorSubcoreMesh(core_axis_name="c", subcore_axis_name="s")

    @functools.partial(
        pl.kernel,
        mesh=mesh,
        out_type=jax.ShapeDtypeStruct((b, width), jnp.float32),
        compiler_params=pltpu.CompilerParams(use_tc_tiling_on_sc=False),
        scratch_types=[
            pltpu.VMEM((bpw,), jnp.int32),
            pltpu.VMEM((bpw, width), jnp.float32),
            pltpu.SemaphoreType.DMA,
        ],
    )
    def gather_k(table_hbm, idx_hbm, out_hbm, idx_v, rows_v, sem):
        wid = lax.axis_index("s") * info.num_cores + lax.axis_index("c")
        base = wid * bpw
        pltpu.sync_copy(idx_hbm.at[pl.ds(base, bpw)], idx_v)
        pltpu.async_copy(table_hbm.at[idx_v], rows_v, sem).wait()
        pltpu.sync_copy(rows_v, out_hbm.at[pl.ds(base, bpw)])

    return gather_k(table, idx)


def kernel(query, key_memory, value_memory, k):
    nq = Q // QT
    scores4, mx = pl.pallas_call(
        _score_max_kernel,
        grid=(NB, nq),
        in_specs=[
            pl.BlockSpec((QT, D), lambda bi, qi: (qi, 0)),
            pl.BlockSpec((BK, D), lambda bi, qi: (bi, 0)),
        ],
        out_specs=[
            pl.BlockSpec((QH, SPB, 8, SB), lambda bi, qi: (qi, bi, 0, 0)),
            pl.BlockSpec((1, QT, SPB), lambda bi, qi: (bi, qi, 0)),
        ],
        out_shape=[
            jax.ShapeDtypeStruct((Q // 8, NSB, 8, SB), jnp.float32),
            jax.ShapeDtypeStruct((NB, Q, SPB), jnp.float32),
        ],
    )(query, key_memory)

    sb, rows = pl.pallas_call(
        _select_blocks_kernel,
        grid=(nq,),
        in_specs=[pl.BlockSpec((NB, QT, SPB), lambda qi: (0, qi, 0))],
        out_specs=[
            pl.BlockSpec((QT, CW), lambda qi: (qi, 0)),
            pl.BlockSpec((QT, CW), lambda qi: (qi, 0)),
        ],
        out_shape=[
            jax.ShapeDtypeStruct((Q, CW), jnp.int32),
            jax.ShapeDtypeStruct((Q, CW), jnp.int32),
        ],
    )(mx)

    idx2 = (rows[:, :TOPK].reshape(Q // 8, 8, TOPK)
            .transpose(0, 2, 1).reshape(-1))
    gathered = _sc_gather(scores4.reshape(Q * NSB, SB), idx2, SB)

    ts, ti = pl.pallas_call(
        _final_topk_kernel,
        grid=(nq,),
        in_specs=[
            pl.BlockSpec((QH, TOPK, 8, SB), lambda qi: (qi, 0, 0, 0)),
            pl.BlockSpec((QT, CW), lambda qi: (qi, 0)),
        ],
        out_specs=[
            pl.BlockSpec((QT, CW), lambda qi: (qi, 0)),
            pl.BlockSpec((QT, CW), lambda qi: (qi, 0)),
        ],
        out_shape=[
            jax.ShapeDtypeStruct((Q, CW), jnp.float32),
            jax.ShapeDtypeStruct((Q, CW), jnp.int32),
        ],
    )(gathered.reshape(Q // 8, TOPK, 8, SB), sb)

    rows_out = _sc_gather(value_memory, ti[:, :TOPK].reshape(-1), D)
    return rows_out.reshape(Q, TOPK, D), ts[:, :TOPK]


# speedup vs baseline: 1.9369x; 1.2631x over previous
import functools

import jax
import jax.numpy as jnp
from jax import lax
from jax.experimental import pallas as pl
from jax.experimental.pallas import tpu as pltpu
from jax.experimental.pallas import tpu_sc as plsc

Q = 1024
K = 100000
D = 64
TOPK = 10

QT = 128
QH = QT // 8
BK = 4096
NB = (K + BK - 1) // BK
KPAD = NB * BK
SB = 128
SPB = BK // SB
NSB = KPAD // SB
CW = 16
GC = TOPK * SB

NEG_INF = float("-inf")
I32_MAX = 2**31 - 1


def _score_max_kernel(q_ref, k_ref, s_ref, mx_ref):
    bi = pl.program_id(0)
    q = q_ref[...]
    kb = k_ref[...]
    s = lax.dot_general(q, kb, (((1,), (1,)), ((), ())),
                        preferred_element_type=jnp.float32)
    col = lax.broadcasted_iota(jnp.int32, (QT, BK), 1)
    s = jnp.where(col + bi * BK < K, s, NEG_INF)
    mxs = []
    for j in range(SPB):
        sj = s[:, j * SB:(j + 1) * SB]
        s_ref[:, j] = sj.reshape(QH, 8, SB)
        mxs.append(jnp.max(sj, axis=1, keepdims=True))
    mx_ref[0] = jnp.concatenate(mxs, axis=1)


def _select_blocks_kernel(mx_ref, sb_ref, row_ref):
    qi = pl.program_id(0)
    c = jnp.concatenate([mx_ref[b] for b in range(NB)], axis=1)
    g = lax.broadcasted_iota(jnp.int32, (QT, NSB), 1)
    qglob = lax.broadcasted_iota(jnp.int32, (QT, 1), 0) + qi * QT
    rbase = (qglob >> 3) * (NSB * 8) + (qglob & 7)
    out_sb, out_row = [], []
    for _ in range(TOPK):
        m = jnp.max(c, axis=1, keepdims=True)
        hit = c == m
        a = jnp.min(jnp.where(hit, g, I32_MAX), axis=1, keepdims=True)
        out_sb.append(a)
        out_row.append(rbase + a * 8)
        c = jnp.where(g == a, NEG_INF, c)
    pad = jnp.zeros((QT, CW - TOPK), jnp.int32)
    sb_ref[...] = jnp.concatenate(out_sb + [pad], axis=1)
    row_ref[...] = jnp.concatenate(out_row + [pad], axis=1)


def _final_topk_kernel(g4_ref, sb_ref, os_ref, oi_ref):
    c = jnp.concatenate(
        [g4_ref[:, j].reshape(QT, SB) for j in range(TOPK)], axis=1)
    sb = sb_ref[...]
    off = lax.broadcasted_iota(jnp.int32, (QT, SB), 1)
    g = jnp.concatenate(
        [sb[:, j:j + 1] * SB + off for j in range(TOPK)], axis=1)
    out_s, out_i = [], []
    for _ in range(TOPK):
        m = jnp.max(c, axis=1, keepdims=True)
        hit = c == m
        a = jnp.min(jnp.where(hit, g, I32_MAX), axis=1, keepdims=True)
        out_s.append(m)
        out_i.append(a)
        c = jnp.where(hit & (g == a), NEG_INF, c)
    pad_s = jnp.full((QT, CW - TOPK), NEG_INF, jnp.float32)
    pad_i = jnp.zeros((QT, CW - TOPK), jnp.int32)
    os_ref[...] = jnp.concatenate(out_s + [pad_s], axis=1)
    oi_ref[...] = jnp.concatenate(out_i + [pad_i], axis=1)


def _sc_gather(table, idx, width):
    info = plsc.get_sparse_core_info()
    nw = info.num_cores * info.num_subcores
    b = idx.shape[0]
    bpw = b // nw
    mesh = plsc.VectorSubcoreMesh(core_axis_name="c", subcore_axis_name="s")

    @functools.partial(
        pl.kernel,
        mesh=mesh,
        out_type=jax.ShapeDtypeStruct((b, width), jnp.float32),
        compiler_params=pltpu.CompilerParams(use_tc_tiling_on_sc=False),
        scratch_types=[
            pltpu.VMEM((bpw,), jnp.int32),
            pltpu.VMEM((bpw, width), jnp.float32),
            pltpu.SemaphoreType.DMA,
        ],
    )
    def gather_k(table_hbm, idx_hbm, out_hbm, idx_v, rows_v, sem):
        wid = lax.axis_index("s") * info.num_cores + lax.axis_index("c")
        base = wid * bpw
        pltpu.sync_copy(idx_hbm.at[pl.ds(base, bpw)], idx_v)
        pltpu.async_copy(table_hbm.at[idx_v], rows_v, sem).wait()
        pltpu.sync_copy(rows_v, out_hbm.at[pl.ds(base, bpw)])

    return gather_k(table, idx)


def kernel(query, key_memory, value_memory, k):
    nq = Q // QT
    scores4, mx = pl.pallas_call(
        _score_max_kernel,
        grid=(NB, nq),
        in_specs=[
            pl.BlockSpec((QT, D), lambda bi, qi: (qi, 0)),
            pl.BlockSpec((BK, D), lambda bi, qi: (bi, 0)),
        ],
        out_specs=[
            pl.BlockSpec((QH, SPB, 8, SB), lambda bi, qi: (qi, bi, 0, 0)),
            pl.BlockSpec((1, QT, SPB), lambda bi, qi: (bi, qi, 0)),
        ],
        out_shape=[
            jax.ShapeDtypeStruct((Q // 8, NSB, 8, SB), jnp.float32),
            jax.ShapeDtypeStruct((NB, Q, SPB), jnp.float32),
        ],
    )(query, key_memory)

    sb, rows = pl.pallas_call(
        _select_blocks_kernel,
        grid=(nq,),
        in_specs=[pl.BlockSpec((NB, QT, SPB), lambda qi: (0, qi, 0))],
        out_specs=[
            pl.BlockSpec((QT, CW), lambda qi: (qi, 0)),
            pl.BlockSpec((QT, CW), lambda qi: (qi, 0)),
        ],
        out_shape=[
            jax.ShapeDtypeStruct((Q, CW), jnp.int32),
            jax.ShapeDtypeStruct((Q, CW), jnp.int32),
        ],
    )(mx)

    idx2 = (rows[:, :TOPK].reshape(Q // 8, 8, TOPK)
            .transpose(0, 2, 1).reshape(-1))
    gathered = _sc_gather(scores4.reshape(Q * NSB, SB), idx2, SB)

    ts, ti = pl.pallas_call(
        _final_topk_kernel,
        grid=(nq,),
        in_specs=[
            pl.BlockSpec((QH, TOPK, 8, SB), lambda qi: (qi, 0, 0, 0)),
            pl.BlockSpec((QT, CW), lambda qi: (qi, 0)),
        ],
        out_specs=[
            pl.BlockSpec((QT, CW), lambda qi: (qi, 0)),
            pl.BlockSpec((QT, CW), lambda qi: (qi, 0)),
        ],
        out_shape=[
            jax.ShapeDtypeStruct((Q, CW), jnp.float32),
            jax.ShapeDtypeStruct((Q, CW), jnp.int32),
        ],
    )(gathered.reshape(Q // 8, TOPK, 8, SB), sb)

    rows_out = _sc_gather(value_memory, ti[:, :TOPK].reshape(-1), D)
    return rows_out.reshape(Q, TOPK, D), ts[:, :TOPK]


# speedup vs baseline: 2.1834x; 1.1273x over previous
import functools

import jax
import jax.numpy as jnp
from jax import lax
from jax.experimental import pallas as pl
from jax.experimental.pallas import tpu as pltpu
from jax.experimental.pallas import tpu_sc as plsc

Q = 1024
K = 100000
D = 64
TOPK = 10

QT = 128
QH = QT // 8
BK = 6272
NB = (K + BK - 1) // BK
KPAD = NB * BK
SB = 128
SPB = BK // SB
NSB = KPAD // SB
CW = 16
GC = TOPK * SB

NEG_INF = float("-inf")
I32_MAX = 2**31 - 1


def _score_max_kernel(q_ref, k_ref, s_ref, mx_ref):
    bi = pl.program_id(0)
    q = q_ref[...]
    kb = k_ref[...]
    s = lax.dot_general(q, kb, (((1,), (1,)), ((), ())),
                        preferred_element_type=jnp.float32)
    col = lax.broadcasted_iota(jnp.int32, (QT, BK), 1)
    s = jnp.where(col + bi * BK < K, s, NEG_INF)
    mxs = []
    for j in range(SPB):
        sj = s[:, j * SB:(j + 1) * SB]
        s_ref[:, j] = sj.reshape(QH, 8, SB)
        mxs.append(jnp.max(sj, axis=1, keepdims=True))
    mx_ref[0] = jnp.concatenate(mxs, axis=1)


def _select_blocks_kernel(mx_ref, sb_ref, row_ref):
    qi = pl.program_id(0)
    c = jnp.concatenate([mx_ref[b] for b in range(NB)], axis=1)
    g = lax.broadcasted_iota(jnp.int32, (QT, NSB), 1)
    qglob = lax.broadcasted_iota(jnp.int32, (QT, 1), 0) + qi * QT
    rbase = (qglob >> 3) * (NSB * 8) + (qglob & 7)
    out_sb, out_row = [], []
    for _ in range(TOPK):
        m = jnp.max(c, axis=1, keepdims=True)
        hit = c == m
        a = jnp.min(jnp.where(hit, g, I32_MAX), axis=1, keepdims=True)
        out_sb.append(a)
        out_row.append(rbase + a * 8)
        c = jnp.where(g == a, NEG_INF, c)
    pad = jnp.zeros((QT, CW - TOPK), jnp.int32)
    sb_ref[...] = jnp.concatenate(out_sb + [pad], axis=1)
    row_ref[...] = jnp.concatenate(out_row + [pad], axis=1)


def _final_topk_kernel(g4_ref, sb_ref, os_ref, oi_ref):
    c = jnp.concatenate(
        [g4_ref[:, j].reshape(QT, SB) for j in range(TOPK)], axis=1)
    sb = sb_ref[...]
    off = lax.broadcasted_iota(jnp.int32, (QT, SB), 1)
    g = jnp.concatenate(
        [sb[:, j:j + 1] * SB + off for j in range(TOPK)], axis=1)
    out_s, out_i = [], []
    for _ in range(TOPK):
        m = jnp.max(c, axis=1, keepdims=True)
        hit = c == m
        a = jnp.min(jnp.where(hit, g, I32_MAX), axis=1, keepdims=True)
        out_s.append(m)
        out_i.append(a)
        c = jnp.where(hit & (g == a), NEG_INF, c)
    pad_s = jnp.full((QT, CW - TOPK), NEG_INF, jnp.float32)
    pad_i = jnp.zeros((QT, CW - TOPK), jnp.int32)
    os_ref[...] = jnp.concatenate(out_s + [pad_s], axis=1)
    oi_ref[...] = jnp.concatenate(out_i + [pad_i], axis=1)


def _sc_gather(table, idx, width):
    info = plsc.get_sparse_core_info()
    nw = info.num_cores * info.num_subcores
    b = idx.shape[0]
    bpw = b // nw
    mesh = plsc.VectorSubcoreMesh(core_axis_name="c", subcore_axis_name="s")

    @functools.partial(
        pl.kernel,
        mesh=mesh,
        out_type=jax.ShapeDtypeStruct((b, width), jnp.float32),
        compiler_params=pltpu.CompilerParams(use_tc_tiling_on_sc=False),
        scratch_types=[
            pltpu.VMEM((bpw,), jnp.int32),
            pltpu.VMEM((bpw, width), jnp.float32),
            pltpu.SemaphoreType.DMA,
        ],
    )
    def gather_k(table_hbm, idx_hbm, out_hbm, idx_v, rows_v, sem):
        wid = lax.axis_index("s") * info.num_cores + lax.axis_index("c")
        base = wid * bpw
        pltpu.sync_copy(idx_hbm.at[pl.ds(base, bpw)], idx_v)
        pltpu.async_copy(table_hbm.at[idx_v], rows_v, sem).wait()
        pltpu.sync_copy(rows_v, out_hbm.at[pl.ds(base, bpw)])

    return gather_k(table, idx)


def kernel(query, key_memory, value_memory, k):
    nq = Q // QT
    scores4, mx = pl.pallas_call(
        _score_max_kernel,
        grid=(NB, nq),
        in_specs=[
            pl.BlockSpec((QT, D), lambda bi, qi: (qi, 0)),
            pl.BlockSpec((BK, D), lambda bi, qi: (bi, 0)),
        ],
        out_specs=[
            pl.BlockSpec((QH, SPB, 8, SB), lambda bi, qi: (qi, bi, 0, 0)),
            pl.BlockSpec((1, QT, SPB), lambda bi, qi: (bi, qi, 0)),
        ],
        out_shape=[
            jax.ShapeDtypeStruct((Q // 8, NSB, 8, SB), jnp.float32),
            jax.ShapeDtypeStruct((NB, Q, SPB), jnp.float32),
        ],
    )(query, key_memory)

    sb, rows = pl.pallas_call(
        _select_blocks_kernel,
        grid=(nq,),
        in_specs=[pl.BlockSpec((NB, QT, SPB), lambda qi: (0, qi, 0))],
        out_specs=[
            pl.BlockSpec((QT, CW), lambda qi: (qi, 0)),
            pl.BlockSpec((QT, CW), lambda qi: (qi, 0)),
        ],
        out_shape=[
            jax.ShapeDtypeStruct((Q, CW), jnp.int32),
            jax.ShapeDtypeStruct((Q, CW), jnp.int32),
        ],
    )(mx)

    idx2 = (rows[:, :TOPK].reshape(Q // 8, 8, TOPK)
            .transpose(0, 2, 1).reshape(-1))
    gathered = _sc_gather(scores4.reshape(Q * NSB, SB), idx2, SB)

    ts, ti = pl.pallas_call(
        _final_topk_kernel,
        grid=(nq,),
        in_specs=[
            pl.BlockSpec((QH, TOPK, 8, SB), lambda qi: (qi, 0, 0, 0)),
            pl.BlockSpec((QT, CW), lambda qi: (qi, 0)),
        ],
        out_specs=[
            pl.BlockSpec((QT, CW), lambda qi: (qi, 0)),
            pl.BlockSpec((QT, CW), lambda qi: (qi, 0)),
        ],
        out_shape=[
            jax.ShapeDtypeStruct((Q, CW), jnp.float32),
            jax.ShapeDtypeStruct((Q, CW), jnp.int32),
        ],
    )(gathered.reshape(Q // 8, TOPK, 8, SB), sb)

    rows_out = _sc_gather(value_memory, ti[:, :TOPK].reshape(-1), D)
    return rows_out.reshape(Q, TOPK, D), ts[:, :TOPK]


# speedup vs baseline: 2.4526x; 1.1233x over previous
import functools

import jax
import jax.numpy as jnp
from jax import lax
from jax.experimental import pallas as pl
from jax.experimental.pallas import tpu as pltpu
from jax.experimental.pallas import tpu_sc as plsc

Q = 1024
K = 100000
D = 64
TOPK = 10

QT = 128
QH = QT // 8
BK = 12544
NB = (K + BK - 1) // BK
KPAD = NB * BK
SB = 128
SPB = BK // SB
NSB = KPAD // SB
CW = 16
GC = TOPK * SB

NEG_INF = float("-inf")
I32_MAX = 2**31 - 1


def _score_max_kernel(q_ref, k_ref, s_ref, mx_ref):
    bi = pl.program_id(0)
    q = q_ref[...]
    kb = k_ref[...]
    s = lax.dot_general(q, kb, (((1,), (1,)), ((), ())),
                        preferred_element_type=jnp.float32)
    col = lax.broadcasted_iota(jnp.int32, (QT, BK), 1)
    s = jnp.where(col + bi * BK < K, s, NEG_INF)
    mxs = []
    for j in range(SPB):
        sj = s[:, j * SB:(j + 1) * SB]
        s_ref[:, j] = sj.reshape(QH, 8, SB)
        mxs.append(jnp.max(sj, axis=1, keepdims=True))
    mx_ref[0] = jnp.concatenate(mxs, axis=1)


def _select_blocks_kernel(mx_ref, sb_ref, row_ref):
    qi = pl.program_id(0)
    c = jnp.concatenate([mx_ref[b] for b in range(NB)], axis=1)
    g = lax.broadcasted_iota(jnp.int32, (QT, NSB), 1)
    qglob = lax.broadcasted_iota(jnp.int32, (QT, 1), 0) + qi * QT
    rbase = (qglob >> 3) * (NSB * 8) + (qglob & 7)
    out_sb, out_row = [], []
    for _ in range(TOPK):
        m = jnp.max(c, axis=1, keepdims=True)
        hit = c == m
        a = jnp.min(jnp.where(hit, g, I32_MAX), axis=1, keepdims=True)
        out_sb.append(a)
        out_row.append(rbase + a * 8)
        c = jnp.where(g == a, NEG_INF, c)
    pad = jnp.zeros((QT, CW - TOPK), jnp.int32)
    sb_ref[...] = jnp.concatenate(out_sb + [pad], axis=1)
    row_ref[...] = jnp.concatenate(out_row + [pad], axis=1)


def _final_topk_kernel(g4_ref, sb_ref, os_ref, oi_ref):
    c = jnp.concatenate(
        [g4_ref[:, j].reshape(QT, SB) for j in range(TOPK)], axis=1)
    sb = sb_ref[...]
    off = lax.broadcasted_iota(jnp.int32, (QT, SB), 1)
    g = jnp.concatenate(
        [sb[:, j:j + 1] * SB + off for j in range(TOPK)], axis=1)
    out_s, out_i = [], []
    for _ in range(TOPK):
        m = jnp.max(c, axis=1, keepdims=True)
        hit = c == m
        a = jnp.min(jnp.where(hit, g, I32_MAX), axis=1, keepdims=True)
        out_s.append(m)
        out_i.append(a)
        c = jnp.where(hit & (g == a), NEG_INF, c)
    pad_s = jnp.full((QT, CW - TOPK), NEG_INF, jnp.float32)
    pad_i = jnp.zeros((QT, CW - TOPK), jnp.int32)
    os_ref[...] = jnp.concatenate(out_s + [pad_s], axis=1)
    oi_ref[...] = jnp.concatenate(out_i + [pad_i], axis=1)


def _sc_gather(table, idx, width):
    info = plsc.get_sparse_core_info()
    nw = info.num_cores * info.num_subcores
    b = idx.shape[0]
    bpw = b // nw
    mesh = plsc.VectorSubcoreMesh(core_axis_name="c", subcore_axis_name="s")

    @functools.partial(
        pl.kernel,
        mesh=mesh,
        out_type=jax.ShapeDtypeStruct((b, width), jnp.float32),
        compiler_params=pltpu.CompilerParams(use_tc_tiling_on_sc=False),
        scratch_types=[
            pltpu.VMEM((bpw,), jnp.int32),
            pltpu.VMEM((bpw, width), jnp.float32),
            pltpu.SemaphoreType.DMA,
        ],
    )
    def gather_k(table_hbm, idx_hbm, out_hbm, idx_v, rows_v, sem):
        wid = lax.axis_index("s") * info.num_cores + lax.axis_index("c")
        base = wid * bpw
        pltpu.sync_copy(idx_hbm.at[pl.ds(base, bpw)], idx_v)
        pltpu.async_copy(table_hbm.at[idx_v], rows_v, sem).wait()
        pltpu.sync_copy(rows_v, out_hbm.at[pl.ds(base, bpw)])

    return gather_k(table, idx)


def kernel(query, key_memory, value_memory, k):
    nq = Q // QT
    scores4, mx = pl.pallas_call(
        _score_max_kernel,
        grid=(NB, nq),
        in_specs=[
            pl.BlockSpec((QT, D), lambda bi, qi: (qi, 0)),
            pl.BlockSpec((BK, D), lambda bi, qi: (bi, 0)),
        ],
        out_specs=[
            pl.BlockSpec((QH, SPB, 8, SB), lambda bi, qi: (qi, bi, 0, 0)),
            pl.BlockSpec((1, QT, SPB), lambda bi, qi: (bi, qi, 0)),
        ],
        out_shape=[
            jax.ShapeDtypeStruct((Q // 8, NSB, 8, SB), jnp.float32),
            jax.ShapeDtypeStruct((NB, Q, SPB), jnp.float32),
        ],
    )(query, key_memory)

    sb, rows = pl.pallas_call(
        _select_blocks_kernel,
        grid=(nq,),
        in_specs=[pl.BlockSpec((NB, QT, SPB), lambda qi: (0, qi, 0))],
        out_specs=[
            pl.BlockSpec((QT, CW), lambda qi: (qi, 0)),
            pl.BlockSpec((QT, CW), lambda qi: (qi, 0)),
        ],
        out_shape=[
            jax.ShapeDtypeStruct((Q, CW), jnp.int32),
            jax.ShapeDtypeStruct((Q, CW), jnp.int32),
        ],
    )(mx)

    idx2 = (rows[:, :TOPK].reshape(Q // 8, 8, TOPK)
            .transpose(0, 2, 1).reshape(-1))
    gathered = _sc_gather(scores4.reshape(Q * NSB, SB), idx2, SB)

    ts, ti = pl.pallas_call(
        _final_topk_kernel,
        grid=(nq,),
        in_specs=[
            pl.BlockSpec((QH, TOPK, 8, SB), lambda qi: (qi, 0, 0, 0)),
            pl.BlockSpec((QT, CW), lambda qi: (qi, 0)),
        ],
        out_specs=[
            pl.BlockSpec((QT, CW), lambda qi: (qi, 0)),
            pl.BlockSpec((QT, CW), lambda qi: (qi, 0)),
        ],
        out_shape=[
            jax.ShapeDtypeStruct((Q, CW), jnp.float32),
            jax.ShapeDtypeStruct((Q, CW), jnp.int32),
        ],
    )(gathered.reshape(Q // 8, TOPK, 8, SB), sb)

    rows_out = _sc_gather(value_memory, ti[:, :TOPK].reshape(-1), D)
    return rows_out.reshape(Q, TOPK, D), ts[:, :TOPK]


# speedup vs baseline: 2.8668x; 1.1689x over previous
import functools

import jax
import jax.numpy as jnp
from jax import lax
from jax.experimental import pallas as pl
from jax.experimental.pallas import tpu as pltpu
from jax.experimental.pallas import tpu_sc as plsc

Q = 1024
K = 100000
D = 64
TOPK = 10

QT = 128
QH = QT // 8
BK = 12544
NB = (K + BK - 1) // BK
KPAD = NB * BK
SB = 128
SPB = BK // SB
NSB = KPAD // SB
CW = 16
GC = TOPK * SB

NEG_INF = float("-inf")
I32_MAX = 2**31 - 1


def _score_max_kernel(q_ref, k_ref, s_ref, mx_ref):
    bi = pl.program_id(0)
    q = q_ref[...]
    kb = k_ref[...]
    s = lax.dot_general(q, kb, (((1,), (0,)), ((), ())),
                        preferred_element_type=jnp.float32)
    col = lax.broadcasted_iota(jnp.int32, (QT, BK), 1)
    s = jnp.where(col + bi * BK < K, s, NEG_INF)
    mxs = []
    for j in range(SPB):
        sj = s[:, j * SB:(j + 1) * SB]
        s_ref[:, j] = sj.reshape(QH, 8, SB)
        mxs.append(jnp.max(sj, axis=1, keepdims=True))
    mx_ref[0] = jnp.concatenate(mxs, axis=1)


def _select_blocks_kernel(mx_ref, sb_ref, row_ref):
    qi = pl.program_id(0)
    c = jnp.concatenate([mx_ref[b] for b in range(NB)], axis=1)
    g = lax.broadcasted_iota(jnp.int32, (QT, NSB), 1)
    qglob = lax.broadcasted_iota(jnp.int32, (QT, 1), 0) + qi * QT
    rbase = (qglob >> 3) * (NSB * 8) + (qglob & 7)
    out_sb, out_row = [], []
    for _ in range(TOPK):
        m = jnp.max(c, axis=1, keepdims=True)
        hit = c == m
        a = jnp.min(jnp.where(hit, g, I32_MAX), axis=1, keepdims=True)
        out_sb.append(a)
        out_row.append(rbase + a * 8)
        c = jnp.where(g == a, NEG_INF, c)
    pad = jnp.zeros((QT, CW - TOPK), jnp.int32)
    sb_ref[...] = jnp.concatenate(out_sb + [pad], axis=1)
    row_ref[...] = jnp.concatenate(out_row + [pad], axis=1)


def _final_topk_kernel(g4_ref, sb_ref, os_ref, oi_ref):
    c = jnp.concatenate(
        [g4_ref[:, j].reshape(QT, SB) for j in range(TOPK)], axis=1)
    sb = sb_ref[...]
    off = lax.broadcasted_iota(jnp.int32, (QT, SB), 1)
    g = jnp.concatenate(
        [sb[:, j:j + 1] * SB + off for j in range(TOPK)], axis=1)
    out_s, out_i = [], []
    for _ in range(TOPK):
        m = jnp.max(c, axis=1, keepdims=True)
        hit = c == m
        a = jnp.min(jnp.where(hit, g, I32_MAX), axis=1, keepdims=True)
        out_s.append(m)
        out_i.append(a)
        c = jnp.where(hit & (g == a), NEG_INF, c)
    pad_s = jnp.full((QT, CW - TOPK), NEG_INF, jnp.float32)
    pad_i = jnp.zeros((QT, CW - TOPK), jnp.int32)
    os_ref[...] = jnp.concatenate(out_s + [pad_s], axis=1)
    oi_ref[...] = jnp.concatenate(out_i + [pad_i], axis=1)


def _values_t_kernel(g3_ref, out_ref):
    for j in range(TOPK):
        out_ref[j] = g3_ref[j].T


def _sc_gather(table, idx, width):
    info = plsc.get_sparse_core_info()
    nw = info.num_cores * info.num_subcores
    b = idx.shape[0]
    bpw = b // nw
    mesh = plsc.VectorSubcoreMesh(core_axis_name="c", subcore_axis_name="s")

    @functools.partial(
        pl.kernel,
        mesh=mesh,
        out_type=jax.ShapeDtypeStruct((b, width), jnp.float32),
        compiler_params=pltpu.CompilerParams(use_tc_tiling_on_sc=False),
        scratch_types=[
            pltpu.VMEM((bpw,), jnp.int32),
            pltpu.VMEM((bpw, width), jnp.float32),
            pltpu.SemaphoreType.DMA,
        ],
    )
    def gather_k(table_hbm, idx_hbm, out_hbm, idx_v, rows_v, sem):
        wid = lax.axis_index("s") * info.num_cores + lax.axis_index("c")
        base = wid * bpw
        pltpu.sync_copy(idx_hbm.at[pl.ds(base, bpw)], idx_v)
        pltpu.async_copy(table_hbm.at[idx_v], rows_v, sem).wait()
        pltpu.sync_copy(rows_v, out_hbm.at[pl.ds(base, bpw)])

    return gather_k(table, idx)


def kernel(query, key_memory, value_memory, k):
    nq = Q // QT
    scores4, mx = pl.pallas_call(
        _score_max_kernel,
        grid=(NB, nq),
        in_specs=[
            pl.BlockSpec((QT, D), lambda bi, qi: (qi, 0)),
            pl.BlockSpec((D, BK), lambda bi, qi: (0, bi)),
        ],
        out_specs=[
            pl.BlockSpec((QH, SPB, 8, SB), lambda bi, qi: (qi, bi, 0, 0)),
            pl.BlockSpec((1, QT, SPB), lambda bi, qi: (bi, qi, 0)),
        ],
        out_shape=[
            jax.ShapeDtypeStruct((Q // 8, NSB, 8, SB), jnp.float32),
            jax.ShapeDtypeStruct((NB, Q, SPB), jnp.float32),
        ],
    )(query, key_memory.T)

    sb, rows = pl.pallas_call(
        _select_blocks_kernel,
        grid=(nq,),
        in_specs=[pl.BlockSpec((NB, QT, SPB), lambda qi: (0, qi, 0))],
        out_specs=[
            pl.BlockSpec((QT, CW), lambda qi: (qi, 0)),
            pl.BlockSpec((QT, CW), lambda qi: (qi, 0)),
        ],
        out_shape=[
            jax.ShapeDtypeStruct((Q, CW), jnp.int32),
            jax.ShapeDtypeStruct((Q, CW), jnp.int32),
        ],
    )(mx)

    idx2 = (rows[:, :TOPK].reshape(Q // 8, 8, TOPK)
            .transpose(0, 2, 1).reshape(-1))
    gathered = _sc_gather(scores4.reshape(Q * NSB, SB), idx2, SB)

    ts, ti = pl.pallas_call(
        _final_topk_kernel,
        grid=(nq,),
        in_specs=[
            pl.BlockSpec((QH, TOPK, 8, SB), lambda qi: (qi, 0, 0, 0)),
            pl.BlockSpec((QT, CW), lambda qi: (qi, 0)),
        ],
        out_specs=[
            pl.BlockSpec((QT, CW), lambda qi: (qi, 0)),
            pl.BlockSpec((QT, CW), lambda qi: (qi, 0)),
        ],
        out_shape=[
            jax.ShapeDtypeStruct((Q, CW), jnp.float32),
            jax.ShapeDtypeStruct((Q, CW), jnp.int32),
        ],
    )(gathered.reshape(Q // 8, TOPK, 8, SB), sb)

    vidx = ti[:, :TOPK].T.reshape(-1)
    rows_out = _sc_gather(value_memory, vidx, D)

    vt = pl.pallas_call(
        _values_t_kernel,
        grid=(nq,),
        in_specs=[pl.BlockSpec((TOPK, QT, D), lambda qi: (0, qi, 0))],
        out_specs=pl.BlockSpec((TOPK, D, QT), lambda qi: (0, 0, qi)),
        out_shape=jax.ShapeDtypeStruct((TOPK, D, Q), jnp.float32),
    )(rows_out.reshape(TOPK, Q, D))

    return jnp.transpose(vt, (2, 0, 1)), ts[:, :TOPK]


# speedup vs baseline: 3.0751x; 1.0726x over previous
import functools

import jax
import jax.numpy as jnp
from jax import lax
from jax.experimental import pallas as pl
from jax.experimental.pallas import tpu as pltpu
from jax.experimental.pallas import tpu_sc as plsc

Q = 1024
K = 100000
D = 64
TOPK = 10

QT = 128
QH = QT // 8
BK = 12544
NB = (K + BK - 1) // BK
KPAD = NB * BK
SB = 128
SPB = BK // SB
NSB = KPAD // SB
CW = 16
GC = TOPK * SB

NEG_INF = float("-inf")
I32_MAX = 2**31 - 1


def _score_max_kernel(q_ref, k_ref, v_ref, s_ref, mx_ref, vp_ref):
    bi = pl.program_id(0)
    qi = pl.program_id(1)
    qt = q_ref[...]
    kb = k_ref[...]
    s = lax.dot_general(qt, kb, (((0,), (0,)), ((), ())),
                        preferred_element_type=jnp.float32)
    col = lax.broadcasted_iota(jnp.int32, (QT, BK), 1)
    s = jnp.where(col + bi * BK < K, s, NEG_INF)
    mxs = []
    for j in range(SPB):
        sj = s[:, j * SB:(j + 1) * SB]
        s_ref[:, j] = sj.reshape(QH, 8, SB)
        mxs.append(jnp.max(sj, axis=1, keepdims=True))
    mx_ref[0] = jnp.concatenate(mxs, axis=1)

    @pl.when(qi == 0)
    def _():
        vb = v_ref[...]
        ident = (lax.broadcasted_iota(jnp.int32, (D, D), 0) ==
                 lax.broadcasted_iota(jnp.int32, (D, D), 1)
                 ).astype(jnp.float32)
        vp_ref[:, :D] = lax.dot_general(
            vb, ident, (((0,), (0,)), ((), ())),
            preferred_element_type=jnp.float32)


def _select_blocks_kernel(mx_ref, sb_ref, row_ref):
    qi = pl.program_id(0)
    c = jnp.concatenate([mx_ref[b] for b in range(NB)], axis=1)
    g = lax.broadcasted_iota(jnp.int32, (QT, NSB), 1)
    qglob = lax.broadcasted_iota(jnp.int32, (QT, 1), 0) + qi * QT
    rbase = (qglob >> 3) * (NSB * 8) + (qglob & 7)
    out_sb, out_row = [], []
    for _ in range(TOPK):
        m = jnp.max(c, axis=1, keepdims=True)
        hit = c == m
        a = jnp.min(jnp.where(hit, g, I32_MAX), axis=1, keepdims=True)
        out_sb.append(a)
        out_row.append(rbase + a * 8)
        c = jnp.where(g == a, NEG_INF, c)
    pad = jnp.zeros((QT, CW - TOPK), jnp.int32)
    sb_ref[...] = jnp.concatenate(out_sb + [pad], axis=1)
    row_ref[...] = jnp.concatenate(out_row + [pad], axis=1)


def _final_topk_kernel(g4_ref, sb_ref, os_ref, oi_ref):
    c = jnp.concatenate(
        [g4_ref[:, j].reshape(QT, SB) for j in range(TOPK)], axis=1)
    sb = sb_ref[...]
    off = lax.broadcasted_iota(jnp.int32, (QT, SB), 1)
    g = jnp.concatenate(
        [sb[:, j:j + 1] * SB + off for j in range(TOPK)], axis=1)
    out_s, out_i = [], []
    for _ in range(TOPK):
        m = jnp.max(c, axis=1, keepdims=True)
        hit = c == m
        a = jnp.min(jnp.where(hit, g, I32_MAX), axis=1, keepdims=True)
        out_s.append(m)
        out_i.append(a)
        c = jnp.where(hit & (g == a), NEG_INF, c)
    pad_s = jnp.full((QT, CW - TOPK), NEG_INF, jnp.float32)
    pad_i = jnp.zeros((QT, CW - TOPK), jnp.int32)
    os_ref[...] = jnp.concatenate(out_s + [pad_s], axis=1)
    oi_ref[...] = jnp.concatenate(out_i + [pad_i], axis=1)


def _values_t_kernel(g3_ref, out_ref):
    for j in range(TOPK):
        out_ref[j] = g3_ref[j][:, :D].T


def _sc_gather(table, idx, width):
    info = plsc.get_sparse_core_info()
    nw = info.num_cores * info.num_subcores
    b = idx.shape[0]
    bpw = b // nw
    mesh = plsc.VectorSubcoreMesh(core_axis_name="c", subcore_axis_name="s")

    @functools.partial(
        pl.kernel,
        mesh=mesh,
        out_type=jax.ShapeDtypeStruct((b, width), jnp.float32),
        compiler_params=pltpu.CompilerParams(use_tc_tiling_on_sc=False),
        scratch_types=[
            pltpu.VMEM((bpw,), jnp.int32),
            pltpu.VMEM((bpw, width), jnp.float32),
            pltpu.SemaphoreType.DMA,
        ],
    )
    def gather_k(table_hbm, idx_hbm, out_hbm, idx_v, rows_v, sem):
        wid = lax.axis_index("s") * info.num_cores + lax.axis_index("c")
        base = wid * bpw
        pltpu.sync_copy(idx_hbm.at[pl.ds(base, bpw)], idx_v)
        pltpu.async_copy(table_hbm.at[idx_v], rows_v, sem).wait()
        pltpu.sync_copy(rows_v, out_hbm.at[pl.ds(base, bpw)])

    return gather_k(table, idx)


def kernel(query, key_memory, value_memory, k):
    nq = Q // QT
    scores4, mx, vpad = pl.pallas_call(
        _score_max_kernel,
        grid=(NB, nq),
        in_specs=[
            pl.BlockSpec((D, QT), lambda bi, qi: (0, qi)),
            pl.BlockSpec((D, BK), lambda bi, qi: (0, bi)),
            pl.BlockSpec((D, BK), lambda bi, qi: (0, bi)),
        ],
        out_specs=[
            pl.BlockSpec((QH, SPB, 8, SB), lambda bi, qi: (qi, bi, 0, 0)),
            pl.BlockSpec((1, QT, SPB), lambda bi, qi: (bi, qi, 0)),
            pl.BlockSpec((BK, SB), lambda bi, qi: (bi, 0)),
        ],
        out_shape=[
            jax.ShapeDtypeStruct((Q // 8, NSB, 8, SB), jnp.float32),
            jax.ShapeDtypeStruct((NB, Q, SPB), jnp.float32),
            jax.ShapeDtypeStruct((KPAD, SB), jnp.float32),
        ],
    )(query.T, key_memory.T, value_memory.T)

    sb, rows = pl.pallas_call(
        _select_blocks_kernel,
        grid=(nq,),
        in_specs=[pl.BlockSpec((NB, QT, SPB), lambda qi: (0, qi, 0))],
        out_specs=[
            pl.BlockSpec((QT, CW), lambda qi: (qi, 0)),
            pl.BlockSpec((QT, CW), lambda qi: (qi, 0)),
        ],
        out_shape=[
            jax.ShapeDtypeStruct((Q, CW), jnp.int32),
            jax.ShapeDtypeStruct((Q, CW), jnp.int32),
        ],
    )(mx)

    idx2 = (rows[:, :TOPK].reshape(Q // 8, 8, TOPK)
            .transpose(0, 2, 1).reshape(-1))
    gathered = _sc_gather(scores4.reshape(Q * NSB, SB), idx2, SB)

    ts, ti = pl.pallas_call(
        _final_topk_kernel,
        grid=(nq,),
        in_specs=[
            pl.BlockSpec((QH, TOPK, 8, SB), lambda qi: (qi, 0, 0, 0)),
            pl.BlockSpec((QT, CW), lambda qi: (qi, 0)),
        ],
        out_specs=[
            pl.BlockSpec((QT, CW), lambda qi: (qi, 0)),
            pl.BlockSpec((QT, CW), lambda qi: (qi, 0)),
        ],
        out_shape=[
            jax.ShapeDtypeStruct((Q, CW), jnp.float32),
            jax.ShapeDtypeStruct((Q, CW), jnp.int32),
        ],
    )(gathered.reshape(Q // 8, TOPK, 8, SB), sb)

    vidx = ti[:, :TOPK].T.reshape(-1)
    rows_out = _sc_gather(vpad, vidx, SB)

    vt = pl.pallas_call(
        _values_t_kernel,
        grid=(nq,),
        in_specs=[pl.BlockSpec((TOPK, QT, SB), lambda qi: (0, qi, 0))],
        out_specs=pl.BlockSpec((TOPK, D, QT), lambda qi: (0, 0, qi)),
        out_shape=jax.ShapeDtypeStruct((TOPK, D, Q), jnp.float32),
    )(rows_out.reshape(TOPK, Q, SB))

    return jnp.transpose(vt, (2, 0, 1)), ts[:, :TOPK]


# speedup vs baseline: 3.0774x; 1.0008x over previous
import functools

import jax
import jax.numpy as jnp
from jax import lax
from jax.experimental import pallas as pl
from jax.experimental.pallas import tpu as pltpu
from jax.experimental.pallas import tpu_sc as plsc

Q = 1024
K = 100000
D = 64
TOPK = 10

QT = 128
QH = QT // 8
BK = 12544
NB = (K + BK - 1) // BK
KPAD = NB * BK
SB = 128
SPB = BK // SB
NSB = KPAD // SB
CW = 16
GC = TOPK * SB

NEG_INF = float("-inf")
I32_MAX = 2**31 - 1


def _score_max_kernel(q_ref, k_ref, v_ref, s_ref, mx_ref, vp_ref):
    bi = pl.program_id(0)
    qi = pl.program_id(1)
    q = q_ref[...]
    kb = k_ref[...]
    s = lax.dot_general(q, kb, (((1,), (0,)), ((), ())),
                        preferred_element_type=jnp.float32)
    col = lax.broadcasted_iota(jnp.int32, (QT, BK), 1)
    s = jnp.where(col + bi * BK < K, s, NEG_INF)
    mxs = []
    for j in range(SPB):
        sj = s[:, j * SB:(j + 1) * SB]
        s_ref[:, j] = sj.reshape(QH, 8, SB)
        mxs.append(jnp.max(sj, axis=1, keepdims=True))
    mx_ref[0] = jnp.concatenate(mxs, axis=1)

    @pl.when(qi == 0)
    def _():
        vb = v_ref[...]
        ident = (lax.broadcasted_iota(jnp.int32, (D, D), 0) ==
                 lax.broadcasted_iota(jnp.int32, (D, D), 1)
                 ).astype(jnp.float32)
        vp_ref[:, :D] = lax.dot_general(
            vb, ident, (((0,), (0,)), ((), ())),
            preferred_element_type=jnp.float32)


def _select_blocks_kernel(mx_ref, sb_ref, row_ref):
    qi = pl.program_id(0)
    c = jnp.concatenate([mx_ref[b] for b in range(NB)], axis=1)
    g = lax.broadcasted_iota(jnp.int32, (QT, NSB), 1)
    qglob = lax.broadcasted_iota(jnp.int32, (QT, 1), 0) + qi * QT
    rbase = (qglob >> 3) * (NSB * 8) + (qglob & 7)
    out_sb, out_row = [], []
    for _ in range(TOPK):
        m = jnp.max(c, axis=1, keepdims=True)
        hit = c == m
        a = jnp.min(jnp.where(hit, g, I32_MAX), axis=1, keepdims=True)
        out_sb.append(a)
        out_row.append(rbase + a * 8)
        c = jnp.where(g == a, NEG_INF, c)
    pad = jnp.zeros((QT, CW - TOPK), jnp.int32)
    sb_ref[...] = jnp.concatenate(out_sb + [pad], axis=1)
    row_ref[...] = jnp.concatenate(out_row + [pad], axis=1)


def _final_topk_kernel(g4_ref, sb_ref, os_ref, oi_ref):
    c = jnp.concatenate(
        [g4_ref[:, j].reshape(QT, SB) for j in range(TOPK)], axis=1)
    sb = sb_ref[...]
    off = lax.broadcasted_iota(jnp.int32, (QT, SB), 1)
    g = jnp.concatenate(
        [sb[:, j:j + 1] * SB + off for j in range(TOPK)], axis=1)
    out_s, out_i = [], []
    for _ in range(TOPK):
        m = jnp.max(c, axis=1, keepdims=True)
        hit = c == m
        a = jnp.min(jnp.where(hit, g, I32_MAX), axis=1, keepdims=True)
        out_s.append(m)
        out_i.append(a)
        c = jnp.where(hit & (g == a), NEG_INF, c)
    pad_s = jnp.full((QT, CW - TOPK), NEG_INF, jnp.float32)
    pad_i = jnp.zeros((QT, CW - TOPK), jnp.int32)
    os_ref[...] = jnp.concatenate(out_s + [pad_s], axis=1)
    oi_ref[...] = jnp.concatenate(out_i + [pad_i], axis=1)


def _values_t_kernel(g3_ref, out_ref):
    for j in range(TOPK):
        out_ref[j] = g3_ref[j][:, :D].T


def _sc_gather(table, idx, width):
    info = plsc.get_sparse_core_info()
    nw = info.num_cores * info.num_subcores
    b = idx.shape[0]
    bpw = b // nw
    mesh = plsc.VectorSubcoreMesh(core_axis_name="c", subcore_axis_name="s")

    @functools.partial(
        pl.kernel,
        mesh=mesh,
        out_type=jax.ShapeDtypeStruct((b, width), jnp.float32),
        compiler_params=pltpu.CompilerParams(use_tc_tiling_on_sc=False),
        scratch_types=[
            pltpu.VMEM((bpw,), jnp.int32),
            pltpu.VMEM((bpw, width), jnp.float32),
            pltpu.SemaphoreType.DMA,
        ],
    )
    def gather_k(table_hbm, idx_hbm, out_hbm, idx_v, rows_v, sem):
        wid = lax.axis_index("s") * info.num_cores + lax.axis_index("c")
        base = wid * bpw
        pltpu.sync_copy(idx_hbm.at[pl.ds(base, bpw)], idx_v)
        pltpu.async_copy(table_hbm.at[idx_v], rows_v, sem).wait()
        pltpu.sync_copy(rows_v, out_hbm.at[pl.ds(base, bpw)])

    return gather_k(table, idx)


def kernel(query, key_memory, value_memory, k):
    nq = Q // QT
    scores4, mx, vpad = pl.pallas_call(
        _score_max_kernel,
        grid=(NB, nq),
        in_specs=[
            pl.BlockSpec((QT, D), lambda bi, qi: (qi, 0)),
            pl.BlockSpec((D, BK), lambda bi, qi: (0, bi)),
            pl.BlockSpec((D, BK), lambda bi, qi: (0, bi)),
        ],
        out_specs=[
            pl.BlockSpec((QH, SPB, 8, SB), lambda bi, qi: (qi, bi, 0, 0)),
            pl.BlockSpec((1, QT, SPB), lambda bi, qi: (bi, qi, 0)),
            pl.BlockSpec((BK, SB), lambda bi, qi: (bi, 0)),
        ],
        out_shape=[
            jax.ShapeDtypeStruct((Q // 8, NSB, 8, SB), jnp.float32),
            jax.ShapeDtypeStruct((NB, Q, SPB), jnp.float32),
            jax.ShapeDtypeStruct((KPAD, SB), jnp.float32),
        ],
    )(query, key_memory.T, value_memory.T)

    sb, rows = pl.pallas_call(
        _select_blocks_kernel,
        grid=(nq,),
        in_specs=[pl.BlockSpec((NB, QT, SPB), lambda qi: (0, qi, 0))],
        out_specs=[
            pl.BlockSpec((QT, CW), lambda qi: (qi, 0)),
            pl.BlockSpec((QT, CW), lambda qi: (qi, 0)),
        ],
        out_shape=[
            jax.ShapeDtypeStruct((Q, CW), jnp.int32),
            jax.ShapeDtypeStruct((Q, CW), jnp.int32),
        ],
    )(mx)

    idx2 = (rows[:, :TOPK].reshape(Q // 8, 8, TOPK)
            .transpose(0, 2, 1).reshape(-1))
    gathered = _sc_gather(scores4.reshape(Q * NSB, SB), idx2, SB)

    ts, ti = pl.pallas_call(
        _final_topk_kernel,
        grid=(nq,),
        in_specs=[
            pl.BlockSpec((QH, TOPK, 8, SB), lambda qi: (qi, 0, 0, 0)),
            pl.BlockSpec((QT, CW), lambda qi: (qi, 0)),
        ],
        out_specs=[
            pl.BlockSpec((QT, CW), lambda qi: (qi, 0)),
            pl.BlockSpec((QT, CW), lambda qi: (qi, 0)),
        ],
        out_shape=[
            jax.ShapeDtypeStruct((Q, CW), jnp.float32),
            jax.ShapeDtypeStruct((Q, CW), jnp.int32),
        ],
    )(gathered.reshape(Q // 8, TOPK, 8, SB), sb)

    vidx = ti[:, :TOPK].T.reshape(-1)
    rows_out = _sc_gather(vpad, vidx, SB)

    vt = pl.pallas_call(
        _values_t_kernel,
        grid=(nq,),
        in_specs=[pl.BlockSpec((TOPK, QT, SB), lambda qi: (0, qi, 0))],
        out_specs=pl.BlockSpec((TOPK, D, QT), lambda qi: (0, 0, qi)),
        out_shape=jax.ShapeDtypeStruct((TOPK, D, Q), jnp.float32),
    )(rows_out.reshape(TOPK, Q, SB))

    return jnp.transpose(vt, (2, 0, 1)), ts[:, :TOPK]


# speedup vs baseline: 3.0927x; 1.0050x over previous
import functools

import jax
import jax.numpy as jnp
from jax import lax
from jax.experimental import pallas as pl
from jax.experimental.pallas import tpu as pltpu
from jax.experimental.pallas import tpu_sc as plsc

Q = 1024
K = 100000
D = 64
TOPK = 10

QT = 128
QH = QT // 8
BK = 12544
NB = (K + BK - 1) // BK
KPAD = NB * BK
SB = 128
SPB = BK // SB
NSB = KPAD // SB
CW = 16
GC = TOPK * SB

NEG_INF = float("-inf")
I32_MAX = 2**31 - 1


def _score_max_kernel(q_ref, k_ref, v_ref, s_ref, mx_ref, vp_ref):
    bi = pl.program_id(0)
    qi = pl.program_id(1)
    q = q_ref[...]
    kb = k_ref[...]
    s = lax.dot_general(q, kb, (((1,), (0,)), ((), ())),
                        preferred_element_type=jnp.float32)
    col = lax.broadcasted_iota(jnp.int32, (QT, BK), 1)
    s = jnp.where(col + bi * BK < K, s, NEG_INF)
    mxs = []
    for j in range(SPB):
        sj = s[:, j * SB:(j + 1) * SB]
        s_ref[:, j] = sj.reshape(QH, 8, SB)
        mxs.append(jnp.max(sj, axis=1, keepdims=True))
    mx_ref[0] = jnp.concatenate(mxs, axis=1)

    @pl.when(qi == 0)
    def _():
        vb = v_ref[...]
        vp_ref[:, :D] = vb.T


def _select_blocks_kernel(mx_ref, sb_ref, row_ref):
    qi = pl.program_id(0)
    c = jnp.concatenate([mx_ref[b] for b in range(NB)], axis=1)
    g = lax.broadcasted_iota(jnp.int32, (QT, NSB), 1)
    qglob = lax.broadcasted_iota(jnp.int32, (QT, 1), 0) + qi * QT
    rbase = (qglob >> 3) * (NSB * 8) + (qglob & 7)
    out_sb, out_row = [], []
    for _ in range(TOPK):
        m = jnp.max(c, axis=1, keepdims=True)
        hit = c == m
        a = jnp.min(jnp.where(hit, g, I32_MAX), axis=1, keepdims=True)
        out_sb.append(a)
        out_row.append(rbase + a * 8)
        c = jnp.where(g == a, NEG_INF, c)
    pad = jnp.zeros((QT, CW - TOPK), jnp.int32)
    sb_ref[...] = jnp.concatenate(out_sb + [pad], axis=1)
    row_ref[...] = jnp.concatenate(out_row + [pad], axis=1)


def _final_topk_kernel(g4_ref, sb_ref, os_ref, oi_ref):
    c = jnp.concatenate(
        [g4_ref[:, j].reshape(QT, SB) for j in range(TOPK)], axis=1)
    sb = sb_ref[...]
    off = lax.broadcasted_iota(jnp.int32, (QT, SB), 1)
    g = jnp.concatenate(
        [sb[:, j:j + 1] * SB + off for j in range(TOPK)], axis=1)
    out_s, out_i = [], []
    for _ in range(TOPK):
        m = jnp.max(c, axis=1, keepdims=True)
        hit = c == m
        a = jnp.min(jnp.where(hit, g, I32_MAX), axis=1, keepdims=True)
        out_s.append(m)
        out_i.append(a)
        c = jnp.where(hit & (g == a), NEG_INF, c)
    pad_s = jnp.full((QT, CW - TOPK), NEG_INF, jnp.float32)
    pad_i = jnp.zeros((QT, CW - TOPK), jnp.int32)
    os_ref[...] = jnp.concatenate(out_s + [pad_s], axis=1)
    oi_ref[...] = jnp.concatenate(out_i + [pad_i], axis=1)


def _values_t_kernel(g3_ref, out_ref):
    for j in range(TOPK):
        out_ref[j] = g3_ref[j][:, :D].T


def _sc_gather(table, idx, width):
    info = plsc.get_sparse_core_info()
    nw = info.num_cores * info.num_subcores
    b = idx.shape[0]
    bpw = b // nw
    mesh = plsc.VectorSubcoreMesh(core_axis_name="c", subcore_axis_name="s")

    @functools.partial(
        pl.kernel,
        mesh=mesh,
        out_type=jax.ShapeDtypeStruct((b, width), jnp.float32),
        compiler_params=pltpu.CompilerParams(use_tc_tiling_on_sc=False),
        scratch_types=[
            pltpu.VMEM((bpw,), jnp.int32),
            pltpu.VMEM((bpw, width), jnp.float32),
            pltpu.SemaphoreType.DMA,
        ],
    )
    def gather_k(table_hbm, idx_hbm, out_hbm, idx_v, rows_v, sem):
        wid = lax.axis_index("s") * info.num_cores + lax.axis_index("c")
        base = wid * bpw
        pltpu.sync_copy(idx_hbm.at[pl.ds(base, bpw)], idx_v)
        pltpu.async_copy(table_hbm.at[idx_v], rows_v, sem).wait()
        pltpu.sync_copy(rows_v, out_hbm.at[pl.ds(base, bpw)])

    return gather_k(table, idx)


def kernel(query, key_memory, value_memory, k):
    nq = Q // QT
    scores4, mx, vpad = pl.pallas_call(
        _score_max_kernel,
        grid=(NB, nq),
        in_specs=[
            pl.BlockSpec((QT, D), lambda bi, qi: (qi, 0)),
            pl.BlockSpec((D, BK), lambda bi, qi: (0, bi)),
            pl.BlockSpec((D, BK), lambda bi, qi: (0, bi)),
        ],
        out_specs=[
            pl.BlockSpec((QH, SPB, 8, SB), lambda bi, qi: (qi, bi, 0, 0)),
            pl.BlockSpec((1, QT, SPB), lambda bi, qi: (bi, qi, 0)),
            pl.BlockSpec((BK, SB), lambda bi, qi: (bi, 0)),
        ],
        out_shape=[
            jax.ShapeDtypeStruct((Q // 8, NSB, 8, SB), jnp.float32),
            jax.ShapeDtypeStruct((NB, Q, SPB), jnp.float32),
            jax.ShapeDtypeStruct((KPAD, SB), jnp.float32),
        ],
    )(query, key_memory.T, value_memory.T)

    sb, rows = pl.pallas_call(
        _select_blocks_kernel,
        grid=(nq,),
        in_specs=[pl.BlockSpec((NB, QT, SPB), lambda qi: (0, qi, 0))],
        out_specs=[
            pl.BlockSpec((QT, CW), lambda qi: (qi, 0)),
            pl.BlockSpec((QT, CW), lambda qi: (qi, 0)),
        ],
        out_shape=[
            jax.ShapeDtypeStruct((Q, CW), jnp.int32),
            jax.ShapeDtypeStruct((Q, CW), jnp.int32),
        ],
    )(mx)

    idx2 = (rows[:, :TOPK].reshape(Q // 8, 8, TOPK)
            .transpose(0, 2, 1).reshape(-1))
    gathered = _sc_gather(scores4.reshape(Q * NSB, SB), idx2, SB)

    ts, ti = pl.pallas_call(
        _final_topk_kernel,
        grid=(nq,),
        in_specs=[
            pl.BlockSpec((QH, TOPK, 8, SB), lambda qi: (qi, 0, 0, 0)),
            pl.BlockSpec((QT, CW), lambda qi: (qi, 0)),
        ],
        out_specs=[
            pl.BlockSpec((QT, CW), lambda qi: (qi, 0)),
            pl.BlockSpec((QT, CW), lambda qi: (qi, 0)),
        ],
        out_shape=[
            jax.ShapeDtypeStruct((Q, CW), jnp.float32),
            jax.ShapeDtypeStruct((Q, CW), jnp.int32),
        ],
    )(gathered.reshape(Q // 8, TOPK, 8, SB), sb)

    vidx = ti[:, :TOPK].T.reshape(-1)
    rows_out = _sc_gather(vpad, vidx, SB)

    vt = pl.pallas_call(
        _values_t_kernel,
        grid=(nq,),
        in_specs=[pl.BlockSpec((TOPK, QT, SB), lambda qi: (0, qi, 0))],
        out_specs=pl.BlockSpec((TOPK, D, QT), lambda qi: (0, 0, qi)),
        out_shape=jax.ShapeDtypeStruct((TOPK, D, Q), jnp.float32),
    )(rows_out.reshape(TOPK, Q, SB))

    return jnp.transpose(vt, (2, 0, 1)), ts[:, :TOPK]


# speedup vs baseline: 3.1022x; 1.0031x over previous
import functools

import jax
import jax.numpy as jnp
from jax import lax
from jax.experimental import pallas as pl
from jax.experimental.pallas import tpu as pltpu
from jax.experimental.pallas import tpu_sc as plsc

Q = 1024
K = 100000
D = 64
TOPK = 10

QT = 128
QH = QT // 8
BK = 12544
NB = (K + BK - 1) // BK
KPAD = NB * BK
SB = 128
SPB = BK // SB
NSB = KPAD // SB
CW = 16
GC = TOPK * SB

NEG_INF = float("-inf")
I32_MAX = 2**31 - 1


def _score_max_kernel(q_ref, k_ref, v_ref, s_ref, sb_ref, row_ref, vp_ref,
                      mxs_ref):
    bi = pl.program_id(0)
    qi = pl.program_id(1)
    q = q_ref[...]
    kb = k_ref[...]
    s = lax.dot_general(q, kb, (((1,), (0,)), ((), ())),
                        preferred_element_type=jnp.float32)
    col = lax.broadcasted_iota(jnp.int32, (QT, BK), 1)
    s = jnp.where(col + bi * BK < K, s, NEG_INF)
    mxs = []
    for j in range(SPB):
        sj = s[:, j * SB:(j + 1) * SB]
        s_ref[:, j] = sj.reshape(QH, 8, SB)
        mxs.append(jnp.max(sj, axis=1, keepdims=True))
    mxs_ref[bi, pl.ds(qi * QT, QT), :] = (
        jnp.concatenate(mxs, axis=1))

    @pl.when(qi == 0)
    def _():
        vb = v_ref[...]
        vp_ref[:, :D] = vb.T

    @pl.when(bi == NB - 1)
    def _():
        c = jnp.concatenate(
            [mxs_ref[b, pl.ds(qi * QT, QT), :] for b in range(NB)],
            axis=1)
        g = lax.broadcasted_iota(jnp.int32, (QT, NSB), 1)
        qglob = lax.broadcasted_iota(jnp.int32, (QT, 1), 0) + qi * QT
        rbase = (qglob >> 3) * (NSB * 8) + (qglob & 7)
        out_sb, out_row = [], []
        for _ in range(TOPK):
            m = jnp.max(c, axis=1, keepdims=True)
            hit = c == m
            a = jnp.min(jnp.where(hit, g, I32_MAX), axis=1, keepdims=True)
            out_sb.append(a)
            out_row.append(rbase + a * 8)
            c = jnp.where(g == a, NEG_INF, c)
        pad = jnp.zeros((QT, CW - TOPK), jnp.int32)
        sb_ref[...] = jnp.concatenate(out_sb + [pad], axis=1)
        row_ref[...] = jnp.concatenate(out_row + [pad], axis=1)


def _final_topk_kernel(g4_ref, sb_ref, os_ref, oi_ref):
    c = jnp.concatenate(
        [g4_ref[:, j].reshape(QT, SB) for j in range(TOPK)], axis=1)
    sb = sb_ref[...]
    off = lax.broadcasted_iota(jnp.int32, (QT, SB), 1)
    g = jnp.concatenate(
        [sb[:, j:j + 1] * SB + off for j in range(TOPK)], axis=1)
    out_s, out_i = [], []
    for _ in range(TOPK):
        m = jnp.max(c, axis=1, keepdims=True)
        hit = c == m
        a = jnp.min(jnp.where(hit, g, I32_MAX), axis=1, keepdims=True)
        out_s.append(m)
        out_i.append(a)
        c = jnp.where(hit & (g == a), NEG_INF, c)
    pad_s = jnp.full((QT, CW - TOPK), NEG_INF, jnp.float32)
    pad_i = jnp.zeros((QT, CW - TOPK), jnp.int32)
    os_ref[...] = jnp.concatenate(out_s + [pad_s], axis=1)
    oi_ref[...] = jnp.concatenate(out_i + [pad_i], axis=1)


def _values_t_kernel(g3_ref, out_ref):
    for j in range(TOPK):
        out_ref[j] = g3_ref[j][:, :D].T


def _sc_gather(table, idx, width):
    info = plsc.get_sparse_core_info()
    nw = info.num_cores * info.num_subcores
    b = idx.shape[0]
    bpw = b // nw
    mesh = plsc.VectorSubcoreMesh(core_axis_name="c", subcore_axis_name="s")

    @functools.partial(
        pl.kernel,
        mesh=mesh,
        out_type=jax.ShapeDtypeStruct((b, width), jnp.float32),
        compiler_params=pltpu.CompilerParams(use_tc_tiling_on_sc=False),
        scratch_types=[
            pltpu.VMEM((bpw,), jnp.int32),
            pltpu.VMEM((bpw, width), jnp.float32),
            pltpu.SemaphoreType.DMA,
        ],
    )
    def gather_k(table_hbm, idx_hbm, out_hbm, idx_v, rows_v, sem):
        wid = lax.axis_index("s") * info.num_cores + lax.axis_index("c")
        base = wid * bpw
        pltpu.sync_copy(idx_hbm.at[pl.ds(base, bpw)], idx_v)
        pltpu.async_copy(table_hbm.at[idx_v], rows_v, sem).wait()
        pltpu.sync_copy(rows_v, out_hbm.at[pl.ds(base, bpw)])

    return gather_k(table, idx)


def kernel(query, key_memory, value_memory, k):
    nq = Q // QT
    scores4, sb, rows, vpad = pl.pallas_call(
        _score_max_kernel,
        grid=(NB, nq),
        in_specs=[
            pl.BlockSpec((QT, D), lambda bi, qi: (qi, 0)),
            pl.BlockSpec((D, BK), lambda bi, qi: (0, bi)),
            pl.BlockSpec((D, BK), lambda bi, qi: (0, bi)),
        ],
        out_specs=[
            pl.BlockSpec((QH, SPB, 8, SB), lambda bi, qi: (qi, bi, 0, 0)),
            pl.BlockSpec((QT, CW), lambda bi, qi: (qi, 0)),
            pl.BlockSpec((QT, CW), lambda bi, qi: (qi, 0)),
            pl.BlockSpec((BK, SB), lambda bi, qi: (bi, 0)),
        ],
        out_shape=[
            jax.ShapeDtypeStruct((Q // 8, NSB, 8, SB), jnp.float32),
            jax.ShapeDtypeStruct((Q, CW), jnp.int32),
            jax.ShapeDtypeStruct((Q, CW), jnp.int32),
            jax.ShapeDtypeStruct((KPAD, SB), jnp.float32),
        ],
        scratch_shapes=[pltpu.VMEM((NB, Q, SPB), jnp.float32)],
    )(query, key_memory.T, value_memory.T)

    idx2 = (rows[:, :TOPK].reshape(Q // 8, 8, TOPK)
            .transpose(0, 2, 1).reshape(-1))
    gathered = _sc_gather(scores4.reshape(Q * NSB, SB), idx2, SB)

    ts, ti = pl.pallas_call(
        _final_topk_kernel,
        grid=(nq,),
        in_specs=[
            pl.BlockSpec((QH, TOPK, 8, SB), lambda qi: (qi, 0, 0, 0)),
            pl.BlockSpec((QT, CW), lambda qi: (qi, 0)),
        ],
        out_specs=[
            pl.BlockSpec((QT, CW), lambda qi: (qi, 0)),
            pl.BlockSpec((QT, CW), lambda qi: (qi, 0)),
        ],
        out_shape=[
            jax.ShapeDtypeStruct((Q, CW), jnp.float32),
            jax.ShapeDtypeStruct((Q, CW), jnp.int32),
        ],
    )(gathered.reshape(Q // 8, TOPK, 8, SB), sb)

    vidx = ti[:, :TOPK].T.reshape(-1)
    rows_out = _sc_gather(vpad, vidx, SB)

    vt = pl.pallas_call(
        _values_t_kernel,
        grid=(nq,),
        in_specs=[pl.BlockSpec((TOPK, QT, SB), lambda qi: (0, qi, 0))],
        out_specs=pl.BlockSpec((TOPK, D, QT), lambda qi: (0, 0, qi)),
        out_shape=jax.ShapeDtypeStruct((TOPK, D, Q), jnp.float32),
    )(rows_out.reshape(TOPK, Q, SB))

    return jnp.transpose(vt, (2, 0, 1)), ts[:, :TOPK]


# speedup vs baseline: 3.1919x; 1.0289x over previous
import functools

import jax
import jax.numpy as jnp
from jax import lax
from jax.experimental import pallas as pl
from jax.experimental.pallas import tpu as pltpu
from jax.experimental.pallas import tpu_sc as plsc

Q = 1024
K = 100000
D = 64
TOPK = 10

QT = 128
QH = QT // 8
BK = 14336
NB = (K + BK - 1) // BK
KPAD = NB * BK
SB = 128
SPB = BK // SB
NSB = KPAD // SB
CW = 16
GC = TOPK * SB

NEG_INF = float("-inf")
I32_MAX = 2**31 - 1


def _score_max_kernel(q_ref, k_ref, v_ref, s_ref, sb_ref, row_ref, vp_ref,
                      mxs_ref):
    bi = pl.program_id(0)
    qi = pl.program_id(1)
    q = q_ref[...]
    kb = k_ref[...]
    s = lax.dot_general(q, kb, (((1,), (0,)), ((), ())),
                        preferred_element_type=jnp.float32)
    col = lax.broadcasted_iota(jnp.int32, (QT, BK), 1)
    s = jnp.where(col + bi * BK < K, s, NEG_INF)
    mxs = []
    for j in range(SPB):
        sj = s[:, j * SB:(j + 1) * SB]
        s_ref[:, j] = sj.reshape(QH, 8, SB)
        mxs.append(jnp.max(sj, axis=1, keepdims=True))
    mxs_ref[bi, pl.ds(qi * QT, QT), :] = (
        jnp.concatenate(mxs, axis=1))

    @pl.when(qi == 0)
    def _():
        vb = v_ref[...]
        vp_ref[:, :D] = vb.T

    @pl.when(bi == NB - 1)
    def _():
        c = jnp.concatenate(
            [mxs_ref[b, pl.ds(qi * QT, QT), :] for b in range(NB)],
            axis=1)
        g = lax.broadcasted_iota(jnp.int32, (QT, NSB), 1)
        qglob = lax.broadcasted_iota(jnp.int32, (QT, 1), 0) + qi * QT
        rbase = (qglob >> 3) * (NSB * 8) + (qglob & 7)
        out_sb, out_row = [], []
        for _ in range(TOPK):
            m = jnp.max(c, axis=1, keepdims=True)
            hit = c == m
            a = jnp.min(jnp.where(hit, g, I32_MAX), axis=1, keepdims=True)
            out_sb.append(a)
            out_row.append(rbase + a * 8)
            c = jnp.where(g == a, NEG_INF, c)
        pad = jnp.zeros((QT, CW - TOPK), jnp.int32)
        sb_ref[...] = jnp.concatenate(out_sb + [pad], axis=1)
        row_ref[...] = jnp.concatenate(out_row + [pad], axis=1)


def _final_topk_kernel(g4_ref, sb_ref, os_ref, oi_ref):
    c = jnp.concatenate(
        [g4_ref[:, j].reshape(QT, SB) for j in range(TOPK)], axis=1)
    sb = sb_ref[...]
    off = lax.broadcasted_iota(jnp.int32, (QT, SB), 1)
    g = jnp.concatenate(
        [sb[:, j:j + 1] * SB + off for j in range(TOPK)], axis=1)
    out_s, out_i = [], []
    for _ in range(TOPK):
        m = jnp.max(c, axis=1, keepdims=True)
        hit = c == m
        a = jnp.min(jnp.where(hit, g, I32_MAX), axis=1, keepdims=True)
        out_s.append(m)
        out_i.append(a)
        c = jnp.where(hit & (g == a), NEG_INF, c)
    pad_s = jnp.full((QT, CW - TOPK), NEG_INF, jnp.float32)
    pad_i = jnp.zeros((QT, CW - TOPK), jnp.int32)
    os_ref[...] = jnp.concatenate(out_s + [pad_s], axis=1)
    oi_ref[...] = jnp.concatenate(out_i + [pad_i], axis=1)


def _values_t_kernel(g3_ref, out_ref):
    for j in range(TOPK):
        out_ref[j] = g3_ref[j][:, :D].T


def _sc_gather(table, idx, width):
    info = plsc.get_sparse_core_info()
    nw = info.num_cores * info.num_subcores
    b = idx.shape[0]
    bpw = b // nw
    mesh = plsc.VectorSubcoreMesh(core_axis_name="c", subcore_axis_name="s")

    @functools.partial(
        pl.kernel,
        mesh=mesh,
        out_type=jax.ShapeDtypeStruct((b, width), jnp.float32),
        compiler_params=pltpu.CompilerParams(use_tc_tiling_on_sc=False),
        scratch_types=[
            pltpu.VMEM((bpw,), jnp.int32),
            pltpu.VMEM((bpw, width), jnp.float32),
            pltpu.SemaphoreType.DMA,
        ],
    )
    def gather_k(table_hbm, idx_hbm, out_hbm, idx_v, rows_v, sem):
        wid = lax.axis_index("s") * info.num_cores + lax.axis_index("c")
        base = wid * bpw
        pltpu.sync_copy(idx_hbm.at[pl.ds(base, bpw)], idx_v)
        pltpu.async_copy(table_hbm.at[idx_v], rows_v, sem).wait()
        pltpu.sync_copy(rows_v, out_hbm.at[pl.ds(base, bpw)])

    return gather_k(table, idx)


def kernel(query, key_memory, value_memory, k):
    nq = Q // QT
    scores4, sb, rows, vpad = pl.pallas_call(
        _score_max_kernel,
        grid=(NB, nq),
        in_specs=[
            pl.BlockSpec((QT, D), lambda bi, qi: (qi, 0)),
            pl.BlockSpec((D, BK), lambda bi, qi: (0, bi)),
            pl.BlockSpec((D, BK), lambda bi, qi: (0, bi)),
        ],
        out_specs=[
            pl.BlockSpec((QH, SPB, 8, SB), lambda bi, qi: (qi, bi, 0, 0)),
            pl.BlockSpec((QT, CW), lambda bi, qi: (qi, 0)),
            pl.BlockSpec((QT, CW), lambda bi, qi: (qi, 0)),
            pl.BlockSpec((BK, SB), lambda bi, qi: (bi, 0)),
        ],
        out_shape=[
            jax.ShapeDtypeStruct((Q // 8, NSB, 8, SB), jnp.float32),
            jax.ShapeDtypeStruct((Q, CW), jnp.int32),
            jax.ShapeDtypeStruct((Q, CW), jnp.int32),
            jax.ShapeDtypeStruct((KPAD, SB), jnp.float32),
        ],
        scratch_shapes=[pltpu.VMEM((NB, Q, SPB), jnp.float32)],
    )(query, key_memory.T, value_memory.T)

    idx2 = (rows[:, :TOPK].reshape(Q // 8, 8, TOPK)
            .transpose(0, 2, 1).reshape(-1))
    gathered = _sc_gather(scores4.reshape(Q * NSB, SB), idx2, SB)

    ts, ti = pl.pallas_call(
        _final_topk_kernel,
        grid=(nq,),
        in_specs=[
            pl.BlockSpec((QH, TOPK, 8, SB), lambda qi: (qi, 0, 0, 0)),
            pl.BlockSpec((QT, CW), lambda qi: (qi, 0)),
        ],
        out_specs=[
            pl.BlockSpec((QT, CW), lambda qi: (qi, 0)),
            pl.BlockSpec((QT, CW), lambda qi: (qi, 0)),
        ],
        out_shape=[
            jax.ShapeDtypeStruct((Q, CW), jnp.float32),
            jax.ShapeDtypeStruct((Q, CW), jnp.int32),
        ],
    )(gathered.reshape(Q // 8, TOPK, 8, SB), sb)

    vidx = ti[:, :TOPK].T.reshape(-1)
    rows_out = _sc_gather(vpad, vidx, SB)

    vt = pl.pallas_call(
        _values_t_kernel,
        grid=(nq,),
        in_specs=[pl.BlockSpec((TOPK, QT, SB), lambda qi: (0, qi, 0))],
        out_specs=pl.BlockSpec((TOPK, D, QT), lambda qi: (0, 0, qi)),
        out_shape=jax.ShapeDtypeStruct((TOPK, D, Q), jnp.float32),
    )(rows_out.reshape(TOPK, Q, SB))

    return jnp.transpose(vt, (2, 0, 1)), ts[:, :TOPK]


# speedup vs baseline: 3.2337x; 1.0131x over previous
import functools

import jax
import jax.numpy as jnp
from jax import lax
from jax.experimental import pallas as pl
from jax.experimental.pallas import tpu as pltpu
from jax.experimental.pallas import tpu_sc as plsc

Q = 1024
K = 100000
D = 64
TOPK = 10

QT = 128
QH = QT // 8
BK = 16768
NB = (K + BK - 1) // BK
KPAD = NB * BK
SB = 128
SPB = BK // SB
NSB = KPAD // SB
CW = 16
GC = TOPK * SB

NEG_INF = float("-inf")
I32_MAX = 2**31 - 1


def _score_max_kernel(q_ref, k_ref, v_ref, s_ref, sb_ref, row_ref, vp_ref,
                      mxs_ref):
    bi = pl.program_id(0)
    qi = pl.program_id(1)
    q = q_ref[...]
    kb = k_ref[...]
    s = lax.dot_general(q, kb, (((1,), (0,)), ((), ())),
                        preferred_element_type=jnp.float32)
    col = lax.broadcasted_iota(jnp.int32, (QT, BK), 1)
    s = jnp.where(col + bi * BK < K, s, NEG_INF)
    mxs = []
    for j in range(SPB):
        sj = s[:, j * SB:(j + 1) * SB]
        s_ref[:, j] = sj.reshape(QH, 8, SB)
        mxs.append(jnp.max(sj, axis=1, keepdims=True))
    mxs_ref[bi, pl.ds(qi * QT, QT), :] = (
        jnp.concatenate(mxs, axis=1))

    @pl.when(qi == 0)
    def _():
        vb = v_ref[...]
        vp_ref[:, :D] = vb.T

    @pl.when(bi == NB - 1)
    def _():
        c = jnp.concatenate(
            [mxs_ref[b, pl.ds(qi * QT, QT), :] for b in range(NB)],
            axis=1)
        g = lax.broadcasted_iota(jnp.int32, (QT, NSB), 1)
        qglob = lax.broadcasted_iota(jnp.int32, (QT, 1), 0) + qi * QT
        rbase = (qglob >> 3) * (NSB * 8) + (qglob & 7)
        out_sb, out_row = [], []
        for _ in range(TOPK):
            m = jnp.max(c, axis=1, keepdims=True)
            hit = c == m
            a = jnp.min(jnp.where(hit, g, I32_MAX), axis=1, keepdims=True)
            out_sb.append(a)
            out_row.append(rbase + a * 8)
            c = jnp.where(g == a, NEG_INF, c)
        pad = jnp.zeros((QT, CW - TOPK), jnp.int32)
        sb_ref[...] = jnp.concatenate(out_sb + [pad], axis=1)
        row_ref[...] = jnp.concatenate(out_row + [pad], axis=1)


def _final_topk_kernel(g4_ref, sb_ref, os_ref, oi_ref):
    c = jnp.concatenate(
        [g4_ref[:, j].reshape(QT, SB) for j in range(TOPK)], axis=1)
    sb = sb_ref[...]
    off = lax.broadcasted_iota(jnp.int32, (QT, SB), 1)
    g = jnp.concatenate(
        [sb[:, j:j + 1] * SB + off for j in range(TOPK)], axis=1)
    out_s, out_i = [], []
    for _ in range(TOPK):
        m = jnp.max(c, axis=1, keepdims=True)
        hit = c == m
        a = jnp.min(jnp.where(hit, g, I32_MAX), axis=1, keepdims=True)
        out_s.append(m)
        out_i.append(a)
        c = jnp.where(hit & (g == a), NEG_INF, c)
    pad_s = jnp.full((QT, CW - TOPK), NEG_INF, jnp.float32)
    pad_i = jnp.zeros((QT, CW - TOPK), jnp.int32)
    os_ref[...] = jnp.concatenate(out_s + [pad_s], axis=1)
    oi_ref[...] = jnp.concatenate(out_i + [pad_i], axis=1)


def _values_t_kernel(g3_ref, out_ref):
    for j in range(TOPK):
        out_ref[j] = g3_ref[j][:, :D].T


def _sc_gather(table, idx, width):
    info = plsc.get_sparse_core_info()
    nw = info.num_cores * info.num_subcores
    b = idx.shape[0]
    bpw = b // nw
    mesh = plsc.VectorSubcoreMesh(core_axis_name="c", subcore_axis_name="s")

    @functools.partial(
        pl.kernel,
        mesh=mesh,
        out_type=jax.ShapeDtypeStruct((b, width), jnp.float32),
        compiler_params=pltpu.CompilerParams(use_tc_tiling_on_sc=False),
        scratch_types=[
            pltpu.VMEM((bpw,), jnp.int32),
            pltpu.VMEM((bpw, width), jnp.float32),
            pltpu.SemaphoreType.DMA,
        ],
    )
    def gather_k(table_hbm, idx_hbm, out_hbm, idx_v, rows_v, sem):
        wid = lax.axis_index("s") * info.num_cores + lax.axis_index("c")
        base = wid * bpw
        pltpu.sync_copy(idx_hbm.at[pl.ds(base, bpw)], idx_v)
        pltpu.async_copy(table_hbm.at[idx_v], rows_v, sem).wait()
        pltpu.sync_copy(rows_v, out_hbm.at[pl.ds(base, bpw)])

    return gather_k(table, idx)


def kernel(query, key_memory, value_memory, k):
    nq = Q // QT
    scores4, sb, rows, vpad = pl.pallas_call(
        _score_max_kernel,
        grid=(NB, nq),
        in_specs=[
            pl.BlockSpec((QT, D), lambda bi, qi: (qi, 0)),
            pl.BlockSpec((D, BK), lambda bi, qi: (0, bi)),
            pl.BlockSpec((D, BK), lambda bi, qi: (0, bi)),
        ],
        out_specs=[
            pl.BlockSpec((QH, SPB, 8, SB), lambda bi, qi: (qi, bi, 0, 0)),
            pl.BlockSpec((QT, CW), lambda bi, qi: (qi, 0)),
            pl.BlockSpec((QT, CW), lambda bi, qi: (qi, 0)),
            pl.BlockSpec((BK, SB), lambda bi, qi: (bi, 0)),
        ],
        out_shape=[
            jax.ShapeDtypeStruct((Q // 8, NSB, 8, SB), jnp.float32),
            jax.ShapeDtypeStruct((Q, CW), jnp.int32),
            jax.ShapeDtypeStruct((Q, CW), jnp.int32),
            jax.ShapeDtypeStruct((KPAD, SB), jnp.float32),
        ],
        scratch_shapes=[pltpu.VMEM((NB, Q, SPB), jnp.float32)],
    )(query, key_memory.T, value_memory.T)

    idx2 = (rows[:, :TOPK].reshape(Q // 8, 8, TOPK)
            .transpose(0, 2, 1).reshape(-1))
    gathered = _sc_gather(scores4.reshape(Q * NSB, SB), idx2, SB)

    ts, ti = pl.pallas_call(
        _final_topk_kernel,
        grid=(nq,),
        in_specs=[
            pl.BlockSpec((QH, TOPK, 8, SB), lambda qi: (qi, 0, 0, 0)),
            pl.BlockSpec((QT, CW), lambda qi: (qi, 0)),
        ],
        out_specs=[
            pl.BlockSpec((QT, CW), lambda qi: (qi, 0)),
            pl.BlockSpec((QT, CW), lambda qi: (qi, 0)),
        ],
        out_shape=[
            jax.ShapeDtypeStruct((Q, CW), jnp.float32),
            jax.ShapeDtypeStruct((Q, CW), jnp.int32),
        ],
    )(gathered.reshape(Q // 8, TOPK, 8, SB), sb)

    vidx = ti[:, :TOPK].T.reshape(-1)
    rows_out = _sc_gather(vpad, vidx, SB)

    vt = pl.pallas_call(
        _values_t_kernel,
        grid=(nq,),
        in_specs=[pl.BlockSpec((TOPK, QT, SB), lambda qi: (0, qi, 0))],
        out_specs=pl.BlockSpec((TOPK, D, QT), lambda qi: (0, 0, qi)),
        out_shape=jax.ShapeDtypeStruct((TOPK, D, Q), jnp.float32),
    )(rows_out.reshape(TOPK, Q, SB))

    return jnp.transpose(vt, (2, 0, 1)), ts[:, :TOPK]
